# Initial kernel scaffold; baseline (speedup 1.0000x reference)
import jax
import jax.numpy as jnp
from jax.experimental import pallas as pl


def kernel(x, input_pts, x6, pts6, x5, pts5, x4, pts4, x3, pts3, x2, pts2, params):
    raise NotImplementedError("write your pallas kernel here")


# speedup vs baseline: 6.9407x; 6.9407x over previous
import functools

import jax
import jax.numpy as jnp
from jax import lax
from jax.experimental import pallas as pl

PL = 48
NC = 16
DIM = 3
OUT_CH = 13


def _ptconv_body(feat_ref, ptst_ref, pts_ref, q_ref, cen_ref, w1_ref, b1_ref,
                 w2_ref, b2_ref, w3_ref, b3_ref, wm_ref, out_ref, *, K, N, C,
                 Mt, cout):
    f_all = feat_ref[0]
    pts = pts_ref[0]
    pts_t = ptst_ref[0]
    q = q_ref[0]

    d2 = None
    for d in range(DIM):
        diff = q[:, d:d + 1] - pts_t[d:d + 1, :]
        sq = diff * diff
        d2 = sq if d2 is None else d2 + sq

    iota = lax.broadcasted_iota(jnp.int32, (Mt, N), 1)
    prels = []
    fks = []
    d2w = d2
    for _ in range(K):
        mn = jnp.min(d2w, axis=1, keepdims=True)
        sel = jnp.where(d2w == mn, iota, N)
        idxk = jnp.min(sel, axis=1, keepdims=True)
        hot = iota == idxk
        oh = hot.astype(jnp.float32)
        d2w = jnp.where(hot, jnp.float32(jnp.inf), d2w)
        f_k = jnp.dot(oh, f_all, preferred_element_type=jnp.float32)
        p_k = jnp.dot(oh, pts, preferred_element_type=jnp.float32)
        prels.append(p_k - q)
        fks.append(f_k)

    maxr = None
    for pr in prels:
        r2 = jnp.sum(pr * pr, axis=1, keepdims=True)
        maxr = r2 if maxr is None else jnp.maximum(maxr, r2)
    maxi = jnp.sqrt(maxr)
    maxi = jnp.where(maxi == 0.0, 1.0, maxi)

    cen = cen_ref[...]
    w1 = w1_ref[...]
    b1 = b1_ref[...]
    w2 = w2_ref[...]
    b2 = b2_ref[...]
    w3 = w3_ref[...]
    b3 = b3_ref[...]

    fs = jnp.zeros((Mt, NC * C), jnp.float32)
    for k in range(K):
        pn = prels[k] / maxi
        dm = jnp.concatenate(
            [pn[:, d:d + 1] - cen[d:d + 1, :] for d in range(DIM)], axis=1)
        h = jnp.maximum(
            jnp.dot(dm, w1, preferred_element_type=jnp.float32) + b1, 0.0)
        h = jnp.maximum(
            jnp.dot(h, w2, preferred_element_type=jnp.float32) + b2, 0.0)
        dv = jnp.maximum(
            jnp.dot(h, w3, preferred_element_type=jnp.float32) + b3, 0.0)
        a = jnp.concatenate([fks[k]] * NC, axis=1)
        bm = jnp.concatenate(
            [jnp.broadcast_to(dv[:, n:n + 1], (Mt, C)) for n in range(NC)],
            axis=1)
        fs = fs + a * bm

    out = jnp.dot(fs, wm_ref[...], preferred_element_type=jnp.float32)
    out_ref[0] = out * (1.0 / K)


def _ptconv(feat, points, queries, K, p, Mt):
    B, N, C = feat.shape
    M = queries.shape[1]
    cout = p["weight"].shape[2]
    points_t = jnp.transpose(points, (0, 2, 1))
    wmat = jnp.transpose(p["weight"], (1, 0, 2)).reshape(NC * C, cout)
    body = functools.partial(_ptconv_body, K=K, N=N, C=C, Mt=Mt, cout=cout)
    grid = (B, M // Mt)
    full2d = lambda shape: pl.BlockSpec(shape, lambda b, t: (0, 0))
    out = pl.pallas_call(
        body,
        grid=grid,
        in_specs=[
            pl.BlockSpec((1, N, C), lambda b, t: (b, 0, 0)),
            pl.BlockSpec((1, DIM, N), lambda b, t: (b, 0, 0)),
            pl.BlockSpec((1, N, DIM), lambda b, t: (b, 0, 0)),
            pl.BlockSpec((1, Mt, DIM), lambda b, t: (b, t, 0)),
            full2d((DIM, NC)),
            full2d((DIM * NC, 2 * NC)),
            full2d((1, 2 * NC)),
            full2d((2 * NC, NC)),
            full2d((1, NC)),
            full2d((NC, NC)),
            full2d((1, NC)),
            full2d((NC * C, cout)),
        ],
        out_specs=pl.BlockSpec((1, Mt, cout), lambda b, t: (b, t, 0)),
        out_shape=jax.ShapeDtypeStruct((B, M, cout), jnp.float32),
    )(feat, points_t, points, queries, p["centers"], p["l1w"].T,
      p["l1b"].reshape(1, -1), p["l2w"].T, p["l2b"].reshape(1, -1),
      p["l3w"].T, p["l3b"].reshape(1, -1), wmat)
    return out


def _bn_relu_body(x_ref, g_ref, b_ref, out_ref):
    x = x_ref[...]
    m = jnp.mean(x, axis=0, keepdims=True)
    xc = x - m
    v = jnp.mean(xc * xc, axis=0, keepdims=True)
    y = xc / jnp.sqrt(v + 1e-5) * g_ref[...] + b_ref[...]
    out_ref[...] = jnp.maximum(y, 0.0)


def _bn_relu(h, g, b):
    B, M, C = h.shape
    x = h.reshape(B * M, C)
    out = pl.pallas_call(
        _bn_relu_body,
        out_shape=jax.ShapeDtypeStruct((B * M, C), jnp.float32),
    )(x, g.reshape(1, -1), b.reshape(1, -1))
    return out.reshape(B, M, C)


def _bn_stats_body(x_ref, m_ref, v_ref):
    x = x_ref[...]
    m = jnp.mean(x, axis=0, keepdims=True)
    xc = x - m
    m_ref[...] = m
    v_ref[...] = jnp.mean(xc * xc, axis=0, keepdims=True)


def _heads_body(x_ref, m_ref, v_ref, g_ref, b_ref, fw_ref, fb_ref, cw_ref,
                cb_ref, xout_ref, cout_ref):
    xc = x_ref[...] - m_ref[...]
    y = jnp.maximum(
        xc / jnp.sqrt(v_ref[...] + 1e-5) * g_ref[...] + b_ref[...], 0.0)
    xout_ref[...] = (
        jnp.dot(y, fw_ref[...], preferred_element_type=jnp.float32) +
        fb_ref[...])
    cout_ref[...] = (
        jnp.dot(y, cw_ref[...], preferred_element_type=jnp.float32) +
        cb_ref[...])


def _bn_relu_heads(h, g, b, fw, fb, cw, cb):
    B, M, C = h.shape
    R = B * M
    x = h.reshape(R, C)
    mean, var = pl.pallas_call(
        _bn_stats_body,
        out_shape=(
            jax.ShapeDtypeStruct((1, C), jnp.float32),
            jax.ShapeDtypeStruct((1, C), jnp.float32),
        ),
    )(x)
    Rt = 4096
    full2d = lambda shape: pl.BlockSpec(shape, lambda t: (0, 0))
    xout, cout = pl.pallas_call(
        _heads_body,
        grid=(R // Rt,),
        in_specs=[
            pl.BlockSpec((Rt, C), lambda t: (t, 0)),
            full2d((1, C)),
            full2d((1, C)),
            full2d((1, C)),
            full2d((1, C)),
            full2d((C, OUT_CH)),
            full2d((1, OUT_CH)),
            full2d((C, 1)),
            full2d((1, 1)),
        ],
        out_specs=(
            pl.BlockSpec((Rt, OUT_CH), lambda t: (t, 0)),
            pl.BlockSpec((Rt, 1), lambda t: (t, 0)),
        ),
        out_shape=(
            jax.ShapeDtypeStruct((R, OUT_CH), jnp.float32),
            jax.ShapeDtypeStruct((R, 1), jnp.float32),
        ),
    )(x, mean, var, g.reshape(1, -1), b.reshape(1, -1), fw.T,
      fb.reshape(1, -1), cw.T, cb.reshape(1, -1))
    return xout.reshape(B, M, OUT_CH), cout.reshape(B, M, 1)


def kernel(x, input_pts, x6, pts6, x5, pts5, x4, pts4, x3, pts3, x2, pts2,
           params):
    p = params
    h = _ptconv(x6, pts6, pts5, 4, p["cv5d"], Mt=64)
    h = _bn_relu(h, p["bn5d_g"], p["bn5d_b"])
    h = jnp.concatenate([h, x5], axis=2)

    h = _ptconv(h, pts5, pts4, 4, p["cv4d"], Mt=256)
    h = _bn_relu(h, p["bn4d_g"], p["bn4d_b"])
    h = jnp.concatenate([h, x4], axis=2)

    h = _ptconv(h, pts4, pts3, 4, p["cv3d"], Mt=512)
    h = _bn_relu(h, p["bn3d_g"], p["bn3d_b"])
    h = jnp.concatenate([h, x3], axis=2)

    h = _ptconv(h, pts3, pts2, 8, p["cv2d"], Mt=512)
    h = _bn_relu(h, p["bn2d_g"], p["bn2d_b"])
    h = jnp.concatenate([h, x2], axis=2)

    h = _ptconv(h, pts2, input_pts, 8, p["cv1d"], Mt=512)
    xout, cout = _bn_relu_heads(h, p["bn1d_g"], p["bn1d_b"], p["fcout_w"],
                                p["fcout_b"], p["ccout_w"], p["ccout_b"])
    return (xout, cout)
